# baseline (speedup 1.0000x reference)
import jax
import jax.numpy as jnp
from jax.experimental import pallas as pl

_EPS = 1e-06


def _pairwise_loss_kernel(p_ref, t_ref, v_ref, sum_ref, cnt_ref):
    p = p_ref[...]
    t = t_ref[...]
    vf = v_ref[...]
    V = p.shape[0]
    acc_s = jnp.zeros_like(p)
    acc_c = jnp.zeros_like(p)
    for j in range(V):
        pj = p[j:j + 1, :]
        tj = t[j:j + 1, :]
        vj = vf[j:j + 1, :]
        dt = tj - t
        x = (pj - p) * jnp.sign(dt)
        sp = jnp.maximum(x, 0.0) + jnp.log1p(jnp.exp(-jnp.abs(x)))
        m = vf * vj * (jnp.abs(dt) > _EPS).astype(jnp.float32)
        acc_s = acc_s + sp * m
        acc_c = acc_c + m
    sum_ref[...] = jnp.sum(acc_s).reshape(1, 1)
    cnt_ref[...] = jnp.sum(acc_c).reshape(1, 1)


def kernel(pred_severity, target_severity, visit_mask):
    p = pred_severity.T
    t = target_severity.T
    v = visit_mask.T.astype(jnp.float32)
    total, count = pl.pallas_call(
        _pairwise_loss_kernel,
        out_shape=[
            jax.ShapeDtypeStruct((1, 1), jnp.float32),
            jax.ShapeDtypeStruct((1, 1), jnp.float32),
        ],
    )(p, t, v)
    total = total[0, 0]
    count = count[0, 0]
    return jnp.where(count > 0, total / jnp.maximum(count, 1.0),
                     jnp.array(0.0, dtype=jnp.float32))


# speedup vs baseline: 2.0462x; 2.0462x over previous
import jax
import jax.numpy as jnp
from jax.experimental import pallas as pl

_EPS = 1e-06


def _pairwise_loss_kernel(p_ref, t_ref, v_ref, sum_ref, cnt_ref):
    p = p_ref[...]
    t = t_ref[...]
    vf = v_ref[...]
    V, B = p.shape
    row_s = jnp.zeros((1, B), jnp.float32)
    row_c = jnp.zeros((1, B), jnp.float32)
    for j in range(1, V):
        pj = p[j:j + 1, :]
        tj = t[j:j + 1, :]
        vj = vf[j:j + 1, :]
        ps = p[0:j, :]
        ts = t[0:j, :]
        vs = vf[0:j, :]
        dt = tj - ts
        dp = ps - pj
        adp = jnp.abs(dp)
        relu_term = jnp.where(dp * dt < 0.0, adp, 0.0)
        loss = relu_term + jnp.log1p(jnp.exp(-adp))
        m = jnp.where(jnp.abs(dt) > _EPS, vs * vj, 0.0)
        row_s = row_s + jnp.sum(loss * m, axis=0, keepdims=True)
        row_c = row_c + jnp.sum(m, axis=0, keepdims=True)
    sum_ref[...] = jnp.sum(row_s).reshape(1, 1)
    cnt_ref[...] = jnp.sum(row_c).reshape(1, 1)


def kernel(pred_severity, target_severity, visit_mask):
    p = pred_severity.T
    t = target_severity.T
    v = visit_mask.T.astype(jnp.float32)
    total, count = pl.pallas_call(
        _pairwise_loss_kernel,
        out_shape=[
            jax.ShapeDtypeStruct((1, 1), jnp.float32),
            jax.ShapeDtypeStruct((1, 1), jnp.float32),
        ],
    )(p, t, v)
    total = total[0, 0]
    count = count[0, 0]
    return jnp.where(count > 0, total / jnp.maximum(count, 1.0),
                     jnp.array(0.0, dtype=jnp.float32))
